# baseline (device time: 14437 ns/iter reference)
import jax
import jax.numpy as jnp
from jax import lax
from jax.experimental import pallas as pl
from jax.experimental.pallas import tpu as pltpu

CH = 128
N_CH = 8
BIG = 3000


def kernel(x, dest):
    m, n = x.shape
    dest_row = dest.reshape(1, m)

    def body(
        dest_ref, x_ref,
        out_ref,
        xv, dv, sendbuf, recv, outv,
        in_sem, dest_sem, out_sems, send_sems, recv_sems,
    ):
        my_x = lax.axis_index("x")
        me = lax.axis_index("y")
        my_z = lax.axis_index("z")
        peer_id = (my_x, 1 - me, my_z)

        barrier = pltpu.get_barrier_semaphore()
        pl.semaphore_signal(
            barrier, inc=1, device_id=peer_id, device_id_type=pl.DeviceIdType.MESH
        )

        in_copy = pltpu.make_async_copy(x_ref, xv, in_sem)
        in_copy.start()
        dest_copy = pltpu.make_async_copy(dest_ref, dv, dest_sem)
        dest_copy.start()
        dest_copy.wait()

        peer_row = dv[...] != me
        cnt = peer_row.astype(jnp.int32)
        csv = jnp.sum(cnt)
        lane = lax.broadcasted_iota(jnp.int32, (1, m), 1)
        cum = cnt
        k = 1
        while k < m:
            cum = cum + jnp.where(lane >= k, pltpu.roll(cum, k, axis=1), 0)
            k *= 2

        s_out = jnp.where(me == 0, 0, m - csv)
        q_send = jnp.where(peer_row, cum - 1 + s_out, BIG)

        r_row = lax.broadcasted_iota(jnp.int32, (1, m), 1)
        q_loc = jnp.where(peer_row, BIG, r_row - cum + jnp.where(me == 0, 0, csv))

        s_in = jnp.where(me == 0, m - csv, 0)
        k0s = s_out // CH
        k1s = (s_out + csv + CH - 1) // CH
        k0r = s_in // CH
        k1r = (s_in + csv + CH - 1) // CH

        in_copy.wait()
        xb = xv[...].astype(jnp.bfloat16)

        pl.semaphore_wait(barrier, 1)

        rdmas = []
        for i in range(N_CH):
            rdma = pltpu.make_async_remote_copy(
                src_ref=sendbuf.at[pl.ds(i * CH, CH)],
                dst_ref=recv.at[pl.ds(i * CH, CH)],
                send_sem=send_sems.at[i],
                recv_sem=recv_sems.at[i],
                device_id=peer_id,
                device_id_type=pl.DeviceIdType.MESH,
            )
            rdmas.append(rdma)

            @pl.when((i >= k0s) & (i < k1s))
            def _():
                u = lax.broadcasted_iota(jnp.int32, (CH, m), 0) + i * CH
                p_chunk = (u == q_send).astype(jnp.bfloat16)
                sendbuf[pl.ds(i * CH, CH)] = jnp.dot(
                    p_chunk, xb, preferred_element_type=jnp.float32
                ).astype(jnp.bfloat16)
                rdma.start()

        iota_out = lax.broadcasted_iota(jnp.int32, (m, m), 0)
        p_loc = (iota_out == q_loc).astype(jnp.bfloat16)
        out_loc = jnp.dot(p_loc, xb, preferred_element_type=jnp.float32).astype(
            jnp.bfloat16
        )

        j = lax.broadcasted_iota(jnp.int32, (CH, 1), 0)
        out_copies = []
        for i in range(N_CH):

            @pl.when((i >= k0r) & (i < k1r))
            def _():
                rdmas[i].wait_recv()

            ji = j + i * CH
            mask = (ji >= s_in) & (ji < s_in + csv)
            outv[pl.ds(i * CH, CH)] = jnp.where(
                mask,
                recv[pl.ds(i * CH, CH)],
                out_loc[i * CH : (i + 1) * CH],
            )
            out_copy = pltpu.make_async_copy(
                outv.at[pl.ds(i * CH, CH)],
                out_ref.at[pl.ds(i * CH, CH)],
                out_sems.at[i],
            )
            out_copy.start()
            out_copies.append(out_copy)

        for c in out_copies:
            c.wait()
        for i in range(N_CH):

            @pl.when((i >= k0s) & (i < k1s))
            def _():
                rdmas[i].wait_send()

    return pl.pallas_call(
        body,
        out_shape=jax.ShapeDtypeStruct((m, n), jnp.bfloat16),
        in_specs=[
            pl.BlockSpec(memory_space=pl.ANY),
            pl.BlockSpec(memory_space=pl.ANY),
        ],
        out_specs=pl.BlockSpec(memory_space=pl.ANY),
        scratch_shapes=[
            pltpu.VMEM((m, n), jnp.float32),
            pltpu.VMEM((1, m), jnp.int32),
            pltpu.VMEM((m, n), jnp.bfloat16),
            pltpu.VMEM((m, n), jnp.bfloat16),
            pltpu.VMEM((m, n), jnp.bfloat16),
            pltpu.SemaphoreType.DMA,
            pltpu.SemaphoreType.DMA,
            pltpu.SemaphoreType.DMA((N_CH,)),
            pltpu.SemaphoreType.DMA((N_CH,)),
            pltpu.SemaphoreType.DMA((N_CH,)),
        ],
        compiler_params=pltpu.CompilerParams(collective_id=0),
    )(dest_row, x)


# device time: 13345 ns/iter; 1.0818x vs baseline; 1.0818x over previous
import jax
import jax.numpy as jnp
from jax import lax
from jax.experimental import pallas as pl
from jax.experimental.pallas import tpu as pltpu

CH = 128
N_CH = 8
BIG = 3000


def kernel(x, dest):
    m, n = x.shape
    dest_row = dest.reshape(1, m)

    def body(
        dest_ref, x_ref,
        out_ref,
        xv, dv, sendbuf, recv, outv,
        in_sem, dest_sem, out_sems, send_sems, recv_sems,
    ):
        my_x = lax.axis_index("x")
        me = lax.axis_index("y")
        my_z = lax.axis_index("z")
        peer_id = (my_x, 1 - me, my_z)

        barrier = pltpu.get_barrier_semaphore()
        pl.semaphore_signal(
            barrier, inc=1, device_id=peer_id, device_id_type=pl.DeviceIdType.MESH
        )

        in_copy = pltpu.make_async_copy(x_ref, xv, in_sem)
        in_copy.start()
        dest_copy = pltpu.make_async_copy(dest_ref, dv, dest_sem)
        dest_copy.start()
        dest_copy.wait()

        peer_row = dv[...] != me
        cnt = peer_row.astype(jnp.int32)
        csv = jnp.sum(cnt)
        lane = lax.broadcasted_iota(jnp.int32, (1, m), 1)
        cum = cnt
        k = 1
        while k < m:
            cum = cum + jnp.where(lane >= k, pltpu.roll(cum, k, axis=1), 0)
            k *= 2

        s_out = jnp.where(me == 0, 0, m - csv)
        q_send = jnp.where(peer_row, cum - 1 + s_out, BIG)

        r_row = lax.broadcasted_iota(jnp.int32, (1, m), 1)
        q_loc = jnp.where(peer_row, BIG, r_row - cum + jnp.where(me == 0, 0, csv))

        s_in = jnp.where(me == 0, m - csv, 0)
        k0s = s_out // CH
        k1s = (s_out + csv + CH - 1) // CH
        k0r = s_in // CH
        k1r = (s_in + csv + CH - 1) // CH

        in_copy.wait()
        xb = xv[...].astype(jnp.bfloat16)

        pl.semaphore_wait(barrier, 1)

        rdmas = []
        for i in range(N_CH):
            rdma = pltpu.make_async_remote_copy(
                src_ref=sendbuf.at[pl.ds(i * CH, CH)],
                dst_ref=recv.at[pl.ds(i * CH, CH)],
                send_sem=send_sems.at[i],
                recv_sem=recv_sems.at[i],
                device_id=peer_id,
                device_id_type=pl.DeviceIdType.MESH,
            )
            rdmas.append(rdma)

            @pl.when((i >= k0s) & (i < k1s))
            def _():
                u = lax.broadcasted_iota(jnp.int32, (CH, m), 0) + i * CH
                p_chunk = (u == q_send).astype(jnp.bfloat16)
                sendbuf[pl.ds(i * CH, CH)] = jnp.dot(
                    p_chunk, xb, preferred_element_type=jnp.float32
                ).astype(jnp.bfloat16)
                rdma.start()

        iota_out = lax.broadcasted_iota(jnp.int32, (m, m), 0)
        p_loc = (iota_out == q_loc).astype(jnp.bfloat16)
        out_loc = jnp.dot(p_loc, xb, preferred_element_type=jnp.float32).astype(
            jnp.bfloat16
        )

        j = lax.broadcasted_iota(jnp.int32, (CH, 1), 0)
        out_copies = []
        for i in range(N_CH):

            @pl.when((i >= k0r) & (i < k1r))
            def _():
                rdmas[i].wait_recv()

            ji = j + i * CH
            mask = (ji >= s_in) & (ji < s_in + csv)
            outv[pl.ds(i * CH, CH)] = jnp.where(
                mask,
                recv[pl.ds(i * CH, CH)],
                out_loc[i * CH : (i + 1) * CH],
            )
            out_copy = pltpu.make_async_copy(
                outv.at[pl.ds(i * CH, CH)],
                out_ref.at[pl.ds(i * CH, CH)],
                out_sems.at[i],
            )
            out_copy.start()
            out_copies.append(out_copy)

        for c in out_copies:
            c.wait()
        for i in range(N_CH):

            @pl.when((i >= k0s) & (i < k1s))
            def _():
                rdmas[i].wait_send()

    return pl.pallas_call(
        body,
        out_shape=jax.ShapeDtypeStruct((m, n), jnp.bfloat16),
        in_specs=[
            pl.BlockSpec(memory_space=pltpu.MemorySpace.HBM),
            pl.BlockSpec(memory_space=pltpu.MemorySpace.HBM),
        ],
        out_specs=pl.BlockSpec(memory_space=pltpu.MemorySpace.HBM),
        scratch_shapes=[
            pltpu.VMEM((m, n), jnp.float32),
            pltpu.VMEM((1, m), jnp.int32),
            pltpu.VMEM((m, n), jnp.bfloat16),
            pltpu.VMEM((m, n), jnp.bfloat16),
            pltpu.VMEM((m, n), jnp.bfloat16),
            pltpu.SemaphoreType.DMA,
            pltpu.SemaphoreType.DMA,
            pltpu.SemaphoreType.DMA((N_CH,)),
            pltpu.SemaphoreType.DMA((N_CH,)),
            pltpu.SemaphoreType.DMA((N_CH,)),
        ],
        compiler_params=pltpu.CompilerParams(collective_id=0),
    )(
        pltpu.with_memory_space_constraint(dest_row, pltpu.MemorySpace.HBM),
        pltpu.with_memory_space_constraint(x, pltpu.MemorySpace.HBM),
    )


# device time: 13116 ns/iter; 1.1007x vs baseline; 1.0175x over previous
import jax
import jax.numpy as jnp
from jax import lax
from jax.experimental import pallas as pl
from jax.experimental.pallas import tpu as pltpu

CH = 128
N_CH = 8
BIG = 3000


def kernel(x, dest):
    m, n = x.shape
    dest_row = dest.reshape(1, m)

    def body(
        dest_ref, x_ref,
        out_ref,
        xv, dv, sendbuf, recv,
        in_sem, dest_sem, send_sems, recv_sems,
    ):
        my_x = lax.axis_index("x")
        me = lax.axis_index("y")
        my_z = lax.axis_index("z")
        peer_id = (my_x, 1 - me, my_z)

        barrier = pltpu.get_barrier_semaphore()
        pl.semaphore_signal(
            barrier, inc=1, device_id=peer_id, device_id_type=pl.DeviceIdType.MESH
        )

        in_copy = pltpu.make_async_copy(x_ref, xv, in_sem)
        in_copy.start()
        dest_copy = pltpu.make_async_copy(dest_ref, dv, dest_sem)
        dest_copy.start()
        dest_copy.wait()

        peer_row = dv[...] != me
        cnt = peer_row.astype(jnp.int32)
        csv = jnp.sum(cnt)
        lane = lax.broadcasted_iota(jnp.int32, (1, m), 1)
        cum = cnt
        k = 1
        while k < m:
            cum = cum + jnp.where(lane >= k, pltpu.roll(cum, k, axis=1), 0)
            k *= 2

        s_out = jnp.where(me == 0, 0, m - csv)
        q_send = jnp.where(peer_row, cum - 1 + s_out, BIG)

        r_row = lax.broadcasted_iota(jnp.int32, (1, m), 1)
        q_loc = jnp.where(peer_row, BIG, r_row - cum + jnp.where(me == 0, 0, csv))

        s_in = jnp.where(me == 0, m - csv, 0)
        k0s = s_out // CH
        k1s = (s_out + csv + CH - 1) // CH
        k0r = s_in // CH
        k1r = (s_in + csv + CH - 1) // CH

        in_copy.wait()
        xf = xv[...]

        pl.semaphore_wait(barrier, 1)

        rdmas = []
        for i in range(N_CH):
            rdma = pltpu.make_async_remote_copy(
                src_ref=sendbuf.at[pl.ds(i * CH, CH)],
                dst_ref=recv.at[pl.ds(i * CH, CH)],
                send_sem=send_sems.at[i],
                recv_sem=recv_sems.at[i],
                device_id=peer_id,
                device_id_type=pl.DeviceIdType.MESH,
            )
            rdmas.append(rdma)

            @pl.when((i >= k0s) & (i < k1s))
            def _():
                u = lax.broadcasted_iota(jnp.int32, (CH, m), 0) + i * CH
                p_chunk = (u == q_send).astype(jnp.float32)
                sendbuf[pl.ds(i * CH, CH)] = jnp.dot(
                    p_chunk, xf, preferred_element_type=jnp.float32
                ).astype(jnp.bfloat16)
                rdma.start()

        iota_out = lax.broadcasted_iota(jnp.int32, (m, m), 0)
        p_loc = (iota_out == q_loc).astype(jnp.float32)
        out_loc = jnp.dot(p_loc, xf, preferred_element_type=jnp.float32).astype(
            jnp.bfloat16
        )

        j = lax.broadcasted_iota(jnp.int32, (CH, 1), 0)
        for i in range(N_CH):

            @pl.when((i >= k0r) & (i < k1r))
            def _():
                rdmas[i].wait_recv()

            ji = j + i * CH
            mask = (ji >= s_in) & (ji < s_in + csv)
            out_ref[pl.ds(i * CH, CH)] = jnp.where(
                mask,
                recv[pl.ds(i * CH, CH)],
                out_loc[i * CH : (i + 1) * CH],
            )

        for i in range(N_CH):

            @pl.when((i >= k0s) & (i < k1s))
            def _():
                rdmas[i].wait_send()

    return pl.pallas_call(
        body,
        out_shape=jax.ShapeDtypeStruct((m, n), jnp.bfloat16),
        in_specs=[
            pl.BlockSpec(memory_space=pltpu.MemorySpace.HBM),
            pl.BlockSpec(memory_space=pltpu.MemorySpace.HBM),
        ],
        out_specs=pl.BlockSpec(memory_space=pltpu.VMEM),
        scratch_shapes=[
            pltpu.VMEM((m, n), jnp.float32),
            pltpu.VMEM((1, m), jnp.int32),
            pltpu.VMEM((m, n), jnp.bfloat16),
            pltpu.VMEM((m, n), jnp.bfloat16),
            pltpu.SemaphoreType.DMA,
            pltpu.SemaphoreType.DMA,
            pltpu.SemaphoreType.DMA((N_CH,)),
            pltpu.SemaphoreType.DMA((N_CH,)),
        ],
        compiler_params=pltpu.CompilerParams(collective_id=0),
    )(
        pltpu.with_memory_space_constraint(dest_row, pltpu.MemorySpace.HBM),
        pltpu.with_memory_space_constraint(x, pltpu.MemorySpace.HBM),
    )


# device time: 13093 ns/iter; 1.1027x vs baseline; 1.0018x over previous
import jax
import jax.numpy as jnp
from jax import lax
from jax.experimental import pallas as pl
from jax.experimental.pallas import tpu as pltpu

CH = 128
N_CH = 8
BIG = 3000


def kernel(x, dest):
    m, n = x.shape
    dest_row = dest.reshape(1, m)

    def body(
        dest_ref, x_ref,
        out_ref,
        xv, dv, sendbuf, recv,
        in_sem, dest_sem, send_sems, recv_sems,
    ):
        my_x = lax.axis_index("x")
        me = lax.axis_index("y")
        my_z = lax.axis_index("z")
        peer_id = (my_x, 1 - me, my_z)

        barrier = pltpu.get_barrier_semaphore()
        pl.semaphore_signal(
            barrier, inc=1, device_id=peer_id, device_id_type=pl.DeviceIdType.MESH
        )

        in_copy = pltpu.make_async_copy(x_ref, xv, in_sem)
        in_copy.start()
        dest_copy = pltpu.make_async_copy(dest_ref, dv, dest_sem)
        dest_copy.start()
        dest_copy.wait()

        peer_row = dv[...] != me
        cnt = peer_row.astype(jnp.int32)
        csv = jnp.sum(cnt)
        lane = lax.broadcasted_iota(jnp.int32, (1, m), 1)
        cum = cnt
        k = 1
        while k < m:
            cum = cum + jnp.where(lane >= k, pltpu.roll(cum, k, axis=1), 0)
            k *= 2

        s_out = jnp.where(me == 0, 0, m - csv)
        q_send = jnp.where(peer_row, cum - 1 + s_out, BIG)

        r_row = lax.broadcasted_iota(jnp.int32, (1, m), 1)
        q_loc = jnp.where(peer_row, BIG, r_row - cum + jnp.where(me == 0, 0, csv))

        s_in = jnp.where(me == 0, m - csv, 0)
        k0s = s_out // CH
        k1s = (s_out + csv + CH - 1) // CH
        k0r = s_in // CH
        k1r = (s_in + csv + CH - 1) // CH

        in_copy.wait()
        xf = xv[...]

        pl.semaphore_wait(barrier, 1)

        rdmas = []
        for i in range(N_CH):
            rdma = pltpu.make_async_remote_copy(
                src_ref=sendbuf.at[pl.ds(i * CH, CH)],
                dst_ref=recv.at[pl.ds(i * CH, CH)],
                send_sem=send_sems.at[i],
                recv_sem=recv_sems.at[i],
                device_id=peer_id,
                device_id_type=pl.DeviceIdType.MESH,
            )
            rdmas.append(rdma)

            @pl.when((i >= k0s) & (i < k1s))
            def _():
                u = lax.broadcasted_iota(jnp.int32, (CH, m), 0) + i * CH
                p_chunk = (u == q_send).astype(jnp.float32)
                sendbuf[pl.ds(i * CH, CH)] = jnp.dot(
                    p_chunk, xf, preferred_element_type=jnp.float32
                ).astype(jnp.bfloat16)
                rdma.start()

        xb = xf.astype(jnp.bfloat16)
        iota_out = lax.broadcasted_iota(jnp.int32, (m, m), 0)
        p_loc = (iota_out == q_loc).astype(jnp.bfloat16)
        out_loc = jnp.dot(p_loc, xb, preferred_element_type=jnp.float32).astype(
            jnp.bfloat16
        )

        j = lax.broadcasted_iota(jnp.int32, (CH, 1), 0)
        for i in range(N_CH):

            @pl.when((i >= k0r) & (i < k1r))
            def _():
                rdmas[i].wait_recv()

            ji = j + i * CH
            mask = (ji >= s_in) & (ji < s_in + csv)
            out_ref[pl.ds(i * CH, CH)] = jnp.where(
                mask,
                recv[pl.ds(i * CH, CH)],
                out_loc[i * CH : (i + 1) * CH],
            )

        for i in range(N_CH):

            @pl.when((i >= k0s) & (i < k1s))
            def _():
                rdmas[i].wait_send()

    return pl.pallas_call(
        body,
        out_shape=jax.ShapeDtypeStruct((m, n), jnp.bfloat16),
        in_specs=[
            pl.BlockSpec(memory_space=pltpu.MemorySpace.HBM),
            pl.BlockSpec(memory_space=pltpu.MemorySpace.HBM),
        ],
        out_specs=pl.BlockSpec(memory_space=pltpu.VMEM),
        scratch_shapes=[
            pltpu.VMEM((m, n), jnp.float32),
            pltpu.VMEM((1, m), jnp.int32),
            pltpu.VMEM((m, n), jnp.bfloat16),
            pltpu.VMEM((m, n), jnp.bfloat16),
            pltpu.SemaphoreType.DMA,
            pltpu.SemaphoreType.DMA,
            pltpu.SemaphoreType.DMA((N_CH,)),
            pltpu.SemaphoreType.DMA((N_CH,)),
        ],
        compiler_params=pltpu.CompilerParams(collective_id=0),
    )(
        pltpu.with_memory_space_constraint(dest_row, pltpu.MemorySpace.HBM),
        pltpu.with_memory_space_constraint(x, pltpu.MemorySpace.HBM),
    )
